# Initial kernel scaffold; baseline (speedup 1.0000x reference)
import jax
import jax.numpy as jnp
from jax.experimental import pallas as pl


def kernel(feats, edge_index, W_rel, b_rel, W_root):
    raise NotImplementedError("write your pallas kernel here")


# speedup vs baseline: 7.3683x; 7.3683x over previous
import functools

import jax
import jax.numpy as jnp
from jax import lax
from jax.experimental import pallas as pl
from jax.experimental.pallas import tpu as pltpu
from jax.experimental.pallas import tpu_sc as plsc

N_NODES = 10000
N_EDGES = 320000
D = 128

NC = 2
NS = 16
NW = NC * NS
E_PER_TILE = N_EDGES // NW
CHUNK = 80
NCHUNK = E_PER_TILE // CHUNK
N_PAD = 10240
ROWS_PER_TILE = N_PAD // NS

_mesh = plsc.VectorSubcoreMesh(core_axis_name="c", subcore_axis_name="s")


@functools.partial(
    pl.kernel,
    out_type=jax.ShapeDtypeStruct((NC, N_PAD, D), jnp.float32),
    mesh=_mesh,
    scratch_types=[
        pltpu.VMEM((NCHUNK, CHUNK), jnp.int32),
        pltpu.VMEM((NCHUNK, CHUNK), jnp.int32),
        pltpu.VMEM((CHUNK, D), jnp.float32),
        pltpu.VMEM_SHARED((N_PAD, D), jnp.float32),
        pltpu.SemaphoreType.DMA,
    ],
)
def _sc_aggregate(feats_hbm, src_hbm, dst_hbm, zeros_hbm, out_hbm,
                  src_v, dst_v, rows_v, agg_sh, sem):
    c = lax.axis_index("c")
    s = lax.axis_index("s")
    wid = c * NS + s

    pltpu.sync_copy(zeros_hbm, agg_sh.at[pl.ds(s * ROWS_PER_TILE, ROWS_PER_TILE)])
    pltpu.sync_copy(src_hbm.at[wid], src_v)
    pltpu.sync_copy(dst_hbm.at[wid], dst_v)
    plsc.subcore_barrier()

    def body(j, carry):
        pltpu.async_copy(feats_hbm.at[src_v.at[j]], rows_v, sem).wait()
        pltpu.sync_copy(rows_v, agg_sh.at[dst_v.at[j]], add=True)
        return carry

    lax.fori_loop(0, NCHUNK, body, 0)
    plsc.subcore_barrier()

    pltpu.sync_copy(agg_sh.at[pl.ds(s * ROWS_PER_TILE, ROWS_PER_TILE)],
                    out_hbm.at[c, pl.ds(s * ROWS_PER_TILE, ROWS_PER_TILE)])


def _dense_body(p0, p1, x, wrel_t, wroot_t, b, o):
    agg = p0[...] + p1[...]
    o[...] = jnp.maximum(
        jnp.dot(agg, wrel_t[...], preferred_element_type=jnp.float32)
        + jnp.dot(x[...], wroot_t[...], preferred_element_type=jnp.float32)
        + b[...],
        0.0,
    )


_ROWS_BLK = 1000


def _dense(p0, p1, feats, wrel_t, wroot_t, b):
    return pl.pallas_call(
        _dense_body,
        grid=(N_NODES // _ROWS_BLK,),
        in_specs=[
            pl.BlockSpec((_ROWS_BLK, D), lambda i: (i, 0)),
            pl.BlockSpec((_ROWS_BLK, D), lambda i: (i, 0)),
            pl.BlockSpec((_ROWS_BLK, D), lambda i: (i, 0)),
            pl.BlockSpec((D, D), lambda i: (0, 0)),
            pl.BlockSpec((D, D), lambda i: (0, 0)),
            pl.BlockSpec((1, D), lambda i: (0, 0)),
        ],
        out_specs=pl.BlockSpec((_ROWS_BLK, D), lambda i: (i, 0)),
        out_shape=jax.ShapeDtypeStruct((N_NODES, D), jnp.float32),
    )(p0, p1, feats, wrel_t, wroot_t, b)


def kernel(feats, edge_index, W_rel, b_rel, W_root):
    ei = edge_index.astype(jnp.int32)
    src = ei[0].reshape(NW, NCHUNK, CHUNK)
    dst = ei[1].reshape(NW, NCHUNK, CHUNK)
    zeros = jnp.zeros((ROWS_PER_TILE, D), jnp.float32)
    partials = _sc_aggregate(feats, src, dst, zeros)
    return _dense(partials[0, :N_NODES], partials[1, :N_NODES], feats,
                  W_rel.T, W_root.T, b_rel.reshape(1, D))


# speedup vs baseline: 11.0610x; 1.5012x over previous
import functools

import jax
import jax.numpy as jnp
from jax import lax
from jax.experimental import pallas as pl
from jax.experimental.pallas import tpu as pltpu
from jax.experimental.pallas import tpu_sc as plsc

N_NODES = 10000
N_EDGES = 320000
D = 128

NC = 2
NS = 16
NW = NC * NS
E_PER_TILE = N_EDGES // NW
CHUNK = 80
NCHUNK = E_PER_TILE // CHUNK
NBUF = 2
N_PAD = 10112
ROWS_PER_TILE = N_PAD // NS

_mesh = plsc.VectorSubcoreMesh(core_axis_name="c", subcore_axis_name="s")


@functools.partial(
    pl.kernel,
    out_type=jax.ShapeDtypeStruct((NC, N_PAD, D), jnp.float32),
    mesh=_mesh,
    scratch_types=[
        pltpu.VMEM((E_PER_TILE,), jnp.int32),
        pltpu.VMEM((NCHUNK, CHUNK), jnp.int32),
        [pltpu.VMEM((CHUNK, D), jnp.float32) for _ in range(NBUF)],
        [pltpu.SemaphoreType.DMA for _ in range(NBUF)],
        pltpu.VMEM_SHARED((N_PAD, D), jnp.float32),
    ],
)
def _sc_aggregate(feats_hbm, src_hbm, dst_hbm, zeros_hbm, out_hbm,
                  src_v, dst_v, rows, sems, agg_sh):
    c = lax.axis_index("c")
    s = lax.axis_index("s")
    wid = c * NS + s

    pltpu.sync_copy(zeros_hbm, agg_sh.at[pl.ds(s * ROWS_PER_TILE, ROWS_PER_TILE)])
    pltpu.sync_copy(src_hbm.at[pl.ds(wid * E_PER_TILE, E_PER_TILE)], src_v)
    pltpu.sync_copy(dst_hbm.at[wid], dst_v)
    plsc.subcore_barrier()

    def _src(j):
        return src_v.at[pl.ds(j * CHUNK, CHUNK)]

    pltpu.async_copy(feats_hbm.at[_src(0)], rows[0], sems[0]).wait()
    pltpu.sync_copy(rows[0], agg_sh.at[dst_v.at[0]], add=True)
    for b in range(NBUF):
        pltpu.async_copy(feats_hbm.at[_src(1 + b)], rows[b], sems[b])

    def body(g, carry):
        for b in range(NBUF):
            j = 1 + g * NBUF + b
            pltpu.make_async_copy(feats_hbm.at[_src(j)], rows[b],
                                  sems[b]).wait()
            pltpu.sync_copy(rows[b], agg_sh.at[dst_v.at[j]], add=True)
            pltpu.async_copy(feats_hbm.at[_src(j + NBUF)], rows[b], sems[b])
        return carry

    lax.fori_loop(0, (NCHUNK - 1) // NBUF - 1, body, 0)
    for b in range(NBUF):
        j = NCHUNK - NBUF + b
        pltpu.make_async_copy(feats_hbm.at[_src(j)], rows[b], sems[b]).wait()
        pltpu.sync_copy(rows[b], agg_sh.at[dst_v.at[j]], add=True)
    plsc.subcore_barrier()

    pltpu.sync_copy(agg_sh.at[pl.ds(s * ROWS_PER_TILE, ROWS_PER_TILE)],
                    out_hbm.at[c, pl.ds(s * ROWS_PER_TILE, ROWS_PER_TILE)])


def _dense_body(p0, p1, x, wrel_t, wroot_t, b, o):
    agg = p0[...] + p1[...]
    o[...] = jnp.maximum(
        jnp.dot(agg, wrel_t[...], preferred_element_type=jnp.float32)
        + jnp.dot(x[...], wroot_t[...], preferred_element_type=jnp.float32)
        + b[...],
        0.0,
    )


_ROWS_BLK = 1000


def _dense(p0, p1, feats, wrel_t, wroot_t, b):
    return pl.pallas_call(
        _dense_body,
        grid=(N_NODES // _ROWS_BLK,),
        in_specs=[
            pl.BlockSpec((_ROWS_BLK, D), lambda i: (i, 0)),
            pl.BlockSpec((_ROWS_BLK, D), lambda i: (i, 0)),
            pl.BlockSpec((_ROWS_BLK, D), lambda i: (i, 0)),
            pl.BlockSpec((D, D), lambda i: (0, 0)),
            pl.BlockSpec((D, D), lambda i: (0, 0)),
            pl.BlockSpec((1, D), lambda i: (0, 0)),
        ],
        out_specs=pl.BlockSpec((_ROWS_BLK, D), lambda i: (i, 0)),
        out_shape=jax.ShapeDtypeStruct((N_NODES, D), jnp.float32),
    )(p0, p1, feats, wrel_t, wroot_t, b)


def kernel(feats, edge_index, W_rel, b_rel, W_root):
    ei = edge_index.astype(jnp.int32)
    src = ei[0]
    dst = ei[1].reshape(NW, NCHUNK, CHUNK)
    zeros = jnp.zeros((ROWS_PER_TILE, D), jnp.float32)
    partials = _sc_aggregate(feats, src, dst, zeros)
    return _dense(partials[0, :N_NODES], partials[1, :N_NODES], feats,
                  W_rel.T, W_root.T, b_rel.reshape(1, D))
